# initial kernel scaffold (unmeasured)
import jax
import jax.numpy as jnp
from jax import lax
from jax.experimental import pallas as pl
from jax.experimental.pallas import tpu as pltpu

N_DEV = 16
LOG2 = 4
B = 128
D = 128


def kernel(x, Win0, Wout0, Win1, Wout1, Win2, Wout2):
    def body(x_ref, win0, wout0, win1, wout1, win2, wout2,
             out_ref, acc_ref, comm_ref, send_sems, recv_sems):
        my = lax.axis_index("i")

        wins = [win0, win1, win2]
        wouts = [wout0, wout1, wout2]

        for r in range(3):
            if r == 0:
                xcur = x_ref[:, :]
            else:
                xcur = acc_ref[:, :]
            h = jnp.maximum(
                jnp.dot(xcur, wins[r][:, :], preferred_element_type=jnp.float32),
                0.0,
            )
            acc_ref[:, :] = jnp.dot(
                h, wouts[r][:, :], preferred_element_type=jnp.float32
            )

            for s in range(LOG2):
                partner = my ^ (1 << s)
                rdma = pltpu.make_async_remote_copy(
                    src_ref=acc_ref,
                    dst_ref=comm_ref.at[r, s],
                    send_sem=send_sems.at[r, s],
                    recv_sem=recv_sems.at[r, s],
                    device_id=(partner,),
                    device_id_type=pl.DeviceIdType.MESH,
                )
                rdma.start()
                rdma.wait()
                acc_ref[:, :] = acc_ref[:, :] + comm_ref[r, s]

        out_ref[:, :] = acc_ref[pl.ds(my * (B // N_DEV), B // N_DEV), :]

    return pl.pallas_call(
        body,
        out_shape=jax.ShapeDtypeStruct((B // N_DEV, D), jnp.float32),
        in_specs=[pl.BlockSpec(memory_space=pltpu.VMEM)] * 7,
        out_specs=pl.BlockSpec(memory_space=pltpu.VMEM),
        scratch_shapes=[
            pltpu.VMEM((B, D), jnp.float32),
            pltpu.VMEM((3, LOG2, B, D), jnp.float32),
            pltpu.SemaphoreType.DMA((3, LOG2)),
            pltpu.SemaphoreType.DMA((3, LOG2)),
        ],
        compiler_params=pltpu.CompilerParams(collective_id=0),
    )(x, Win0, Wout0, Win1, Wout1, Win2, Wout2)


# baseline (device time: 49886 ns/iter reference)
import jax
import jax.numpy as jnp
from jax import lax
from jax.experimental import pallas as pl
from jax.experimental.pallas import tpu as pltpu

N_DEV = 16
LOG2 = 4
B = 128
D = 128


def kernel(x, Win0, Wout0, Win1, Wout1, Win2, Wout2):
    def body(x_ref, win0, wout0, win1, wout1, win2, wout2,
             out_ref, acc_ref, comm_ref, send_sems, recv_sems):
        my = lax.axis_index("i")

        wins = [win0, win1, win2]
        wouts = [wout0, wout1, wout2]

        for r in range(3):
            if r == 0:
                xcur = x_ref[:, :]
            else:
                xcur = acc_ref[:, :]
            h = jnp.maximum(
                jnp.dot(xcur, wins[r][:, :], preferred_element_type=jnp.float32),
                0.0,
            )
            acc_ref[:, :] = jnp.dot(
                h, wouts[r][:, :], preferred_element_type=jnp.float32
            )

            for s in range(LOG2):
                partner = my ^ (1 << s)
                rdma = pltpu.make_async_remote_copy(
                    src_ref=acc_ref,
                    dst_ref=comm_ref.at[r, s],
                    send_sem=send_sems.at[r, s],
                    recv_sem=recv_sems.at[r, s],
                    device_id=(partner,),
                    device_id_type=pl.DeviceIdType.MESH,
                )
                rdma.start()
                rdma.wait()
                acc_ref[:, :] = acc_ref[:, :] + comm_ref[r, s]

        out_ref[:, :] = acc_ref[pl.ds(my * (B // N_DEV), B // N_DEV), :]

    return pl.pallas_call(
        body,
        out_shape=jax.ShapeDtypeStruct((B // N_DEV, D), jnp.float32),
        in_specs=[pl.BlockSpec(memory_space=pltpu.VMEM)] * 7,
        out_specs=pl.BlockSpec(memory_space=pltpu.VMEM),
        scratch_shapes=[
            pltpu.VMEM((B, D), jnp.float32),
            pltpu.VMEM((3, LOG2, B, D), jnp.float32),
            pltpu.SemaphoreType.DMA((3, LOG2)),
            pltpu.SemaphoreType.DMA((3, LOG2)),
        ],
    )(x, Win0, Wout0, Win1, Wout1, Win2, Wout2)


# device time: 40846 ns/iter; 1.2213x vs baseline; 1.2213x over previous
import jax
import jax.numpy as jnp
from jax import lax
from jax.experimental import pallas as pl
from jax.experimental.pallas import tpu as pltpu

N_DEV = 16
B = 128
D = 128
RPS = B // N_DEV


def kernel(x, Win0, Wout0, Win1, Wout1, Win2, Wout2):
    def body(x_ref, win0, wout0, win1, wout1, win2, wout2,
             out_ref, acc_ref, comm_ref, rs_ref,
             send_sems, recv_sems, rs_send_sems, rs_recv_sems):
        my = lax.axis_index("i")
        j = my % 4
        k = my // 4
        base = my - j

        wins = [win0, win1, win2]
        wouts = [wout0, wout1, wout2]

        def exchange(r, phase, peer_of):
            rdmas = []
            for d in (1, 2, 3):
                rdma = pltpu.make_async_remote_copy(
                    src_ref=acc_ref,
                    dst_ref=comm_ref.at[r, phase, d - 1],
                    send_sem=send_sems.at[r, phase, d - 1],
                    recv_sem=recv_sems.at[r, phase, d - 1],
                    device_id=(peer_of(d),),
                    device_id_type=pl.DeviceIdType.MESH,
                )
                rdma.start()
                rdmas.append(rdma)
            for rdma in rdmas:
                rdma.wait()
            acc_ref[:, :] = (
                acc_ref[:, :]
                + comm_ref[r, phase, 0]
                + comm_ref[r, phase, 1]
                + comm_ref[r, phase, 2]
            )

        for r in range(3):
            xcur = x_ref[:, :] if r == 0 else acc_ref[:, :]
            h = jnp.maximum(
                jnp.dot(xcur, wins[r][:, :], preferred_element_type=jnp.float32),
                0.0,
            )
            acc_ref[:, :] = jnp.dot(
                h, wouts[r][:, :], preferred_element_type=jnp.float32
            )

            exchange(r, 0, lambda d: base + (j + d) % 4)

            if r < 2:
                exchange(r, 1, lambda d: 4 * ((k + d) % 4) + j)
            else:
                rdmas = []
                for d in (1, 2, 3):
                    tgt = 4 * ((k + d) % 4) + j
                    rdma = pltpu.make_async_remote_copy(
                        src_ref=acc_ref.at[pl.ds(tgt * RPS, RPS), :],
                        dst_ref=rs_ref.at[d - 1],
                        send_sem=rs_send_sems.at[d - 1],
                        recv_sem=rs_recv_sems.at[d - 1],
                        device_id=(tgt,),
                        device_id_type=pl.DeviceIdType.MESH,
                    )
                    rdma.start()
                    rdmas.append(rdma)
                for rdma in rdmas:
                    rdma.wait()
                out_ref[:, :] = (
                    acc_ref[pl.ds(my * RPS, RPS), :]
                    + rs_ref[0]
                    + rs_ref[1]
                    + rs_ref[2]
                )

    return pl.pallas_call(
        body,
        out_shape=jax.ShapeDtypeStruct((RPS, D), jnp.float32),
        in_specs=[pl.BlockSpec(memory_space=pltpu.VMEM)] * 7,
        out_specs=pl.BlockSpec(memory_space=pltpu.VMEM),
        scratch_shapes=[
            pltpu.VMEM((B, D), jnp.float32),
            pltpu.VMEM((3, 2, 3, B, D), jnp.float32),
            pltpu.VMEM((3, RPS, D), jnp.float32),
            pltpu.SemaphoreType.DMA((3, 2, 3)),
            pltpu.SemaphoreType.DMA((3, 2, 3)),
            pltpu.SemaphoreType.DMA((3,)),
            pltpu.SemaphoreType.DMA((3,)),
        ],
    )(x, Win0, Wout0, Win1, Wout1, Win2, Wout2)


# device time: 35502 ns/iter; 1.4052x vs baseline; 1.1505x over previous
import jax
import jax.numpy as jnp
from jax import lax
from jax.experimental import pallas as pl
from jax.experimental.pallas import tpu as pltpu

N_DEV = 16
B = 128
D = 128
RPS = B // N_DEV


def kernel(x, Win0, Wout0, Win1, Wout1, Win2, Wout2):
    def body(x_ref, win0, wout0, win1, wout1, win2, wout2,
             out_ref, acc_ref, comm_ref, rs_ref,
             send_sems, recv_sems, rs_send_sems, rs_recv_sems):
        my = lax.axis_index("i")
        j = my % 4
        k = my // 4
        base = my - j

        wins = [win0, win1, win2]
        wouts = [wout0, wout1, wout2]

        barrier_sem = pltpu.get_barrier_semaphore()
        for d in (1, 2, 3):
            for peer in (base + (j + d) % 4, 4 * ((k + d) % 4) + j):
                pl.semaphore_signal(
                    barrier_sem, inc=1,
                    device_id=(peer,), device_id_type=pl.DeviceIdType.MESH,
                )
        pl.semaphore_wait(barrier_sem, 6)

        def exchange(r, phase, peer_of):
            rdmas = []
            for d in (1, 2, 3):
                rdma = pltpu.make_async_remote_copy(
                    src_ref=acc_ref,
                    dst_ref=comm_ref.at[r, phase, d - 1],
                    send_sem=send_sems.at[r, phase, d - 1],
                    recv_sem=recv_sems.at[r, phase, d - 1],
                    device_id=(peer_of(d),),
                    device_id_type=pl.DeviceIdType.MESH,
                )
                rdma.start()
                rdmas.append(rdma)
            for rdma in rdmas:
                rdma.wait()
            acc_ref[:, :] = (
                acc_ref[:, :]
                + comm_ref[r, phase, 0]
                + comm_ref[r, phase, 1]
                + comm_ref[r, phase, 2]
            )

        for r in range(3):
            xcur = x_ref[:, :] if r == 0 else acc_ref[:, :]
            h = jnp.maximum(
                jnp.dot(xcur, wins[r][:, :], preferred_element_type=jnp.float32),
                0.0,
            )
            acc_ref[:, :] = jnp.dot(
                h, wouts[r][:, :], preferred_element_type=jnp.float32
            )

            exchange(r, 0, lambda d: base + (j + d) % 4)

            if r < 2:
                exchange(r, 1, lambda d: 4 * ((k + d) % 4) + j)
            else:
                rdmas = []
                for d in (1, 2, 3):
                    tgt = 4 * ((k + d) % 4) + j
                    rdma = pltpu.make_async_remote_copy(
                        src_ref=acc_ref.at[pl.ds(tgt * RPS, RPS), :],
                        dst_ref=rs_ref.at[d - 1],
                        send_sem=rs_send_sems.at[d - 1],
                        recv_sem=rs_recv_sems.at[d - 1],
                        device_id=(tgt,),
                        device_id_type=pl.DeviceIdType.MESH,
                    )
                    rdma.start()
                    rdmas.append(rdma)
                for rdma in rdmas:
                    rdma.wait()
                out_ref[:, :] = (
                    acc_ref[pl.ds(my * RPS, RPS), :]
                    + rs_ref[0]
                    + rs_ref[1]
                    + rs_ref[2]
                )

    return pl.pallas_call(
        body,
        out_shape=jax.ShapeDtypeStruct((RPS, D), jnp.float32),
        in_specs=[pl.BlockSpec(memory_space=pltpu.VMEM)] * 7,
        out_specs=pl.BlockSpec(memory_space=pltpu.VMEM),
        scratch_shapes=[
            pltpu.VMEM((B, D), jnp.float32),
            pltpu.VMEM((3, 2, 3, B, D), jnp.float32),
            pltpu.VMEM((3, RPS, D), jnp.float32),
            pltpu.SemaphoreType.DMA((3, 2, 3)),
            pltpu.SemaphoreType.DMA((3, 2, 3)),
            pltpu.SemaphoreType.DMA((3,)),
            pltpu.SemaphoreType.DMA((3,)),
        ],
        compiler_params=pltpu.CompilerParams(collective_id=0),
    )(x, Win0, Wout0, Win1, Wout1, Win2, Wout2)


# device time: 7806 ns/iter; 6.3907x vs baseline; 4.5480x over previous
import jax
import jax.numpy as jnp
from jax import lax
from jax.experimental import pallas as pl
from jax.experimental.pallas import tpu as pltpu

N_DEV = 16
B = 128
D = 128
RPS = B // N_DEV


def kernel(x, Win0, Wout0, Win1, Wout1, Win2, Wout2):
    def body(x_ref, win0, wout0, win1, wout1, win2, wout2,
             out_ref, acc_ref, comm_ref, rs_ref,
             send_sems, recv_sems, rs_send_sems, rs_recv_sems):
        my = lax.axis_index("i")
        j = my % 4
        k = my // 4
        base = my - j

        wins = [win0, win1, win2]
        wouts = [wout0, wout1, wout2]

        barrier_sem = pltpu.get_barrier_semaphore()
        for d in (1, 2, 3):
            for peer in (base + (j + d) % 4, 4 * ((k + d) % 4) + j):
                pl.semaphore_signal(
                    barrier_sem, inc=1,
                    device_id=(peer,), device_id_type=pl.DeviceIdType.MESH,
                )
        pl.semaphore_wait(barrier_sem, 6)

        def exchange(r, phase, peer_of):
            rdmas = []
            for d in (1, 2, 3):
                rdma = pltpu.make_async_remote_copy(
                    src_ref=acc_ref,
                    dst_ref=comm_ref.at[r, phase, d - 1],
                    send_sem=send_sems.at[r, phase, d - 1],
                    recv_sem=recv_sems.at[r, phase, d - 1],
                    device_id=(peer_of(d),),
                    device_id_type=pl.DeviceIdType.MESH,
                )
                rdma.start()
                rdmas.append(rdma)
            for rdma in rdmas:
                rdma.wait()
            acc_ref[:, :] = (
                acc_ref[:, :]
                + comm_ref[r, phase, 0]
                + comm_ref[r, phase, 1]
                + comm_ref[r, phase, 2]
            )

        for r in range(3):
            xcur = x_ref[:, :] if r == 0 else acc_ref[:, :]
            h = jnp.maximum(
                jnp.dot(xcur, wins[r][:, :], preferred_element_type=jnp.float32),
                0.0,
            )
            acc_ref[:, :] = jnp.dot(
                h, wouts[r][:, :], preferred_element_type=jnp.float32
            )

            if r < 2:
                exchange(r, 0, lambda d: base + (j + d) % 4)
                exchange(r, 1, lambda d: 4 * ((k + d) % 4) + j)
            else:
                rdmas = []
                for d in range(1, N_DEV):
                    tgt = (my + d) % N_DEV
                    rdma = pltpu.make_async_remote_copy(
                        src_ref=acc_ref.at[pl.ds(tgt * RPS, RPS), :],
                        dst_ref=rs_ref.at[d - 1],
                        send_sem=rs_send_sems.at[d - 1],
                        recv_sem=rs_recv_sems.at[d - 1],
                        device_id=(tgt,),
                        device_id_type=pl.DeviceIdType.MESH,
                    )
                    rdma.start()
                    rdmas.append(rdma)
                for rdma in rdmas:
                    rdma.wait()
                total = acc_ref[pl.ds(my * RPS, RPS), :]
                for d in range(1, N_DEV):
                    total = total + rs_ref[d - 1]
                out_ref[:, :] = total

    return pl.pallas_call(
        body,
        out_shape=jax.ShapeDtypeStruct((RPS, D), jnp.float32),
        in_specs=[pl.BlockSpec(memory_space=pltpu.VMEM)] * 7,
        out_specs=pl.BlockSpec(memory_space=pltpu.VMEM),
        scratch_shapes=[
            pltpu.VMEM((B, D), jnp.float32),
            pltpu.VMEM((3, 2, 3, B, D), jnp.float32),
            pltpu.VMEM((N_DEV - 1, RPS, D), jnp.float32),
            pltpu.SemaphoreType.DMA((3, 2, 3)),
            pltpu.SemaphoreType.DMA((3, 2, 3)),
            pltpu.SemaphoreType.DMA((N_DEV - 1,)),
            pltpu.SemaphoreType.DMA((N_DEV - 1,)),
        ],
        compiler_params=pltpu.CompilerParams(collective_id=0),
    )(x, Win0, Wout0, Win1, Wout1, Win2, Wout2)
